# baseline (speedup 1.0000x reference)
import jax
import jax.numpy as jnp
from jax import lax
from jax.experimental import pallas as pl
from jax.experimental.pallas import tpu as pltpu
from jax.experimental.pallas import tpu_sc as plsc

NUM_ENT = 1000000
NUM_REL = 1000
EMBED_DIM = 64
BATCH = 4096
FEW = 50

NC = 2
NS = 16
NW = NC * NS

BLK = FEW * 2
BPW = BATCH // NW
G = 4
NSB = BPW // G
REL_PER_W = BATCH // NW

TBLK = 1024
TGRID = (NUM_ENT + TBLK - 1) // TBLK


def _tc_transpose_body(tab_t_ref, out_ref):
    out_ref[:, :EMBED_DIM] = tab_t_ref[...].T


def _tc_transpose(tab_t):
    return pl.pallas_call(
        _tc_transpose_body,
        grid=(TGRID,),
        in_specs=[pl.BlockSpec((EMBED_DIM, TBLK), lambda j: (0, j))],
        out_specs=pl.BlockSpec((TBLK, 128), lambda j: (j, 0)),
        out_shape=jax.ShapeDtypeStruct((NUM_ENT, 128), jnp.float32),
    )(tab_t)


def _sc_body(idx_hbm, idxR_hbm, ent_hbm, rel_hbm, ent_out, rel_out,
             idx_v, rows_a, rows_b, ridx_v, rrows_v, sem_a, sem_b, sem_r):
    wid = lax.axis_index("s") * NC + lax.axis_index("c")
    b0w = wid * BPW

    pltpu.sync_copy(idxR_hbm.at[pl.ds(b0w, REL_PER_W)], ridx_v)
    rel_dma = pltpu.async_copy(rel_hbm.at[ridx_v], rrows_v, sem_r)

    pltpu.sync_copy(idx_hbm.at[pl.ds(b0w, BPW)], idx_v)

    rel_dma.wait()
    pltpu.sync_copy(rrows_v, rel_out.at[pl.ds(b0w, REL_PER_W), pl.ds(0, EMBED_DIM)])

    def fire(buf, sem, sb):
        for k in range(G):
            pltpu.async_copy(ent_hbm.at[idx_v.at[sb * G + k]],
                             buf.at[pl.ds(k * BLK, BLK)], sem)

    def drain(buf, sem, sb):
        for k in range(G):
            pltpu.make_async_copy(ent_hbm.at[idx_v.at[sb * G + k]],
                                  buf.at[pl.ds(k * BLK, BLK)], sem).wait()
        pltpu.sync_copy(buf, ent_out.at[pl.ds((b0w + sb * G) * BLK, G * BLK)])

    fire(rows_a, sem_a, 0)

    def pair(j2, carry):
        s0 = 2 * j2
        fire(rows_b, sem_b, s0 + 1)
        drain(rows_a, sem_a, s0)

        @pl.when(s0 + 2 < NSB)
        def _():
            fire(rows_a, sem_a, s0 + 2)

        drain(rows_b, sem_b, s0 + 1)
        return carry

    lax.fori_loop(0, NSB // 2, pair, None)


@jax.jit
def _run(idx2d, idxR1d, ent_table, rel_table):
    t128 = _tc_transpose(ent_table.T)
    mesh = plsc.VectorSubcoreMesh(core_axis_name="c", subcore_axis_name="s",
                                  num_cores=NC, num_subcores=NS)
    kfn = pl.kernel(
        _sc_body,
        compiler_params=pltpu.CompilerParams(use_tc_tiling_on_sc=False),
        out_type=(
            jax.ShapeDtypeStruct((BATCH * FEW * 2, 128), jnp.float32),
            jax.ShapeDtypeStruct((BATCH, 128), jnp.float32),
        ),
        mesh=mesh,
        scratch_types=[
            pltpu.VMEM((BPW, BLK), jnp.int32),
            pltpu.VMEM((G * BLK, 128), jnp.float32),
            pltpu.VMEM((G * BLK, 128), jnp.float32),
            pltpu.VMEM((REL_PER_W,), jnp.int32),
            pltpu.VMEM((REL_PER_W, EMBED_DIM), jnp.float32),
            pltpu.SemaphoreType.DMA,
            pltpu.SemaphoreType.DMA,
            pltpu.SemaphoreType.DMA,
        ],
    )
    return kfn(idx2d, idxR1d, t128, rel_table)


def kernel(idx, idxR, ent_table, rel_table):
    idx2d = idx.reshape(BATCH, FEW * 2).astype(jnp.int32)
    idxR1d = idxR.reshape(BATCH).astype(jnp.int32)
    ent128, rel128 = _run(idx2d, idxR1d, ent_table, rel_table)
    ent_emb = ent128[:, :EMBED_DIM].reshape(BATCH, FEW, 2, EMBED_DIM)
    rel_emb = rel128[:, :EMBED_DIM].reshape(BATCH, 1, 1, EMBED_DIM)
    return (ent_emb, rel_emb)


# speedup vs baseline: 1.6160x; 1.6160x over previous
import jax
import jax.numpy as jnp
from jax import lax
from jax.experimental import pallas as pl
from jax.experimental.pallas import tpu as pltpu
from jax.experimental.pallas import tpu_sc as plsc

NUM_ENT = 1000000
NUM_REL = 1000
EMBED_DIM = 64
BATCH = 4096
FEW = 50

NC = 2
NS = 16
NW = NC * NS

BLK = FEW * 2
BPW = BATCH // NW
G = 4
NSB = BPW // G
REL_PER_W = BATCH // NW

TBLK = 4096
TGRID = (NUM_ENT + TBLK - 1) // TBLK


def _tc_transpose_body(tab_t_ref, out_ref):
    out_ref[:, :EMBED_DIM] = tab_t_ref[...].T


def _tc_transpose(tab_t):
    return pl.pallas_call(
        _tc_transpose_body,
        grid=(TGRID,),
        in_specs=[pl.BlockSpec((EMBED_DIM, TBLK), lambda j: (0, j))],
        out_specs=pl.BlockSpec((TBLK, 128), lambda j: (j, 0)),
        out_shape=jax.ShapeDtypeStruct((NUM_ENT, 128), jnp.float32),
    )(tab_t)


def _sc_body(idx_hbm, idxR_hbm, ent_hbm, rel_hbm, ent_out, rel_out,
             idx_v, rows_a, rows_b, ridx_v, rrows_v, sem_a, sem_b, sem_r):
    wid = lax.axis_index("s") * NC + lax.axis_index("c")
    b0w = wid * BPW

    pltpu.sync_copy(idxR_hbm.at[pl.ds(b0w, REL_PER_W)], ridx_v)
    rel_dma = pltpu.async_copy(rel_hbm.at[ridx_v], rrows_v, sem_r)

    pltpu.sync_copy(idx_hbm.at[pl.ds(b0w, BPW)], idx_v)

    rel_dma.wait()
    pltpu.sync_copy(rrows_v, rel_out.at[pl.ds(b0w, REL_PER_W), pl.ds(0, EMBED_DIM)])

    def fire(buf, sem, sb):
        for k in range(G):
            pltpu.async_copy(ent_hbm.at[idx_v.at[sb * G + k]],
                             buf.at[pl.ds(k * BLK, BLK)], sem)

    def drain(buf, sem, sb):
        for k in range(G):
            pltpu.make_async_copy(ent_hbm.at[idx_v.at[sb * G + k]],
                                  buf.at[pl.ds(k * BLK, BLK)], sem).wait()
        pltpu.sync_copy(buf, ent_out.at[pl.ds((b0w + sb * G) * BLK, G * BLK)])

    fire(rows_a, sem_a, 0)

    def pair(j2, carry):
        s0 = 2 * j2
        fire(rows_b, sem_b, s0 + 1)
        drain(rows_a, sem_a, s0)

        @pl.when(s0 + 2 < NSB)
        def _():
            fire(rows_a, sem_a, s0 + 2)

        drain(rows_b, sem_b, s0 + 1)
        return carry

    lax.fori_loop(0, NSB // 2, pair, None)


@jax.jit
def _run(idx2d, idxR1d, ent_table, rel_table):
    t128 = _tc_transpose(ent_table.T)
    mesh = plsc.VectorSubcoreMesh(core_axis_name="c", subcore_axis_name="s",
                                  num_cores=NC, num_subcores=NS)
    kfn = pl.kernel(
        _sc_body,
        compiler_params=pltpu.CompilerParams(use_tc_tiling_on_sc=False),
        out_type=(
            jax.ShapeDtypeStruct((BATCH * FEW * 2, 128), jnp.float32),
            jax.ShapeDtypeStruct((BATCH, 128), jnp.float32),
        ),
        mesh=mesh,
        scratch_types=[
            pltpu.VMEM((BPW, BLK), jnp.int32),
            pltpu.VMEM((G * BLK, 128), jnp.float32),
            pltpu.VMEM((G * BLK, 128), jnp.float32),
            pltpu.VMEM((REL_PER_W,), jnp.int32),
            pltpu.VMEM((REL_PER_W, EMBED_DIM), jnp.float32),
            pltpu.SemaphoreType.DMA,
            pltpu.SemaphoreType.DMA,
            pltpu.SemaphoreType.DMA,
        ],
    )
    return kfn(idx2d, idxR1d, t128, rel_table)


def kernel(idx, idxR, ent_table, rel_table):
    idx2d = idx.reshape(BATCH, FEW * 2).astype(jnp.int32)
    idxR1d = idxR.reshape(BATCH).astype(jnp.int32)
    ent128, rel128 = _run(idx2d, idxR1d, ent_table, rel_table)
    ent_emb = ent128[:, :EMBED_DIM].reshape(BATCH, FEW, 2, EMBED_DIM)
    rel_emb = rel128[:, :EMBED_DIM].reshape(BATCH, 1, 1, EMBED_DIM)
    return (ent_emb, rel_emb)


# speedup vs baseline: 1.9858x; 1.2289x over previous
import jax
import jax.numpy as jnp
from jax import lax
from jax.experimental import pallas as pl
from jax.experimental.pallas import tpu as pltpu
from jax.experimental.pallas import tpu_sc as plsc

NUM_ENT = 1000000
NUM_REL = 1000
EMBED_DIM = 64
BATCH = 4096
FEW = 50

NC = 2
NS = 16
NW = NC * NS

BLK = FEW * 2
BPW = BATCH // NW
G = 4
NSB = BPW // G
REL_PER_W = BATCH // NW

TBLK = 8192
TGRID = (NUM_ENT + TBLK - 1) // TBLK


def _tc_transpose_body(tab_t_ref, out_ref):
    out_ref[:, :EMBED_DIM] = tab_t_ref[...].T


def _tc_transpose(tab_t):
    return pl.pallas_call(
        _tc_transpose_body,
        grid=(TGRID,),
        in_specs=[pl.BlockSpec((EMBED_DIM, TBLK), lambda j: (0, j))],
        out_specs=pl.BlockSpec((TBLK, 128), lambda j: (j, 0)),
        out_shape=jax.ShapeDtypeStruct((NUM_ENT, 128), jnp.float32),
    )(tab_t)


def _sc_body(idx_hbm, idxR_hbm, ent_hbm, rel_hbm, ent_out, rel_out,
             idx_v, rows_a, rows_b, ridx_v, rrows_v, sem_a, sem_b, sem_r):
    wid = lax.axis_index("s") * NC + lax.axis_index("c")
    b0w = wid * BPW

    pltpu.sync_copy(idxR_hbm.at[pl.ds(b0w, REL_PER_W)], ridx_v)
    rel_dma = pltpu.async_copy(rel_hbm.at[ridx_v], rrows_v, sem_r)

    pltpu.sync_copy(idx_hbm.at[pl.ds(b0w, BPW)], idx_v)

    rel_dma.wait()
    pltpu.sync_copy(rrows_v, rel_out.at[pl.ds(b0w, REL_PER_W), pl.ds(0, EMBED_DIM)])

    def fire(buf, sem, sb):
        for k in range(G):
            pltpu.async_copy(ent_hbm.at[idx_v.at[sb * G + k]],
                             buf.at[pl.ds(k * BLK, BLK)], sem)

    def drain(buf, sem, sb):
        for k in range(G):
            pltpu.make_async_copy(ent_hbm.at[idx_v.at[sb * G + k]],
                                  buf.at[pl.ds(k * BLK, BLK)], sem).wait()
        pltpu.sync_copy(
            buf.at[:, pl.ds(0, EMBED_DIM)],
            ent_out.at[pl.ds((b0w + sb * G) * BLK, G * BLK), pl.ds(0, EMBED_DIM)])

    fire(rows_a, sem_a, 0)

    def pair(j2, carry):
        s0 = 2 * j2
        fire(rows_b, sem_b, s0 + 1)
        drain(rows_a, sem_a, s0)

        @pl.when(s0 + 2 < NSB)
        def _():
            fire(rows_a, sem_a, s0 + 2)

        drain(rows_b, sem_b, s0 + 1)
        return carry

    lax.fori_loop(0, NSB // 2, pair, None)


@jax.jit
def _run(idx2d, idxR1d, ent_table, rel_table):
    t128 = _tc_transpose(ent_table.T)
    mesh = plsc.VectorSubcoreMesh(core_axis_name="c", subcore_axis_name="s",
                                  num_cores=NC, num_subcores=NS)
    kfn = pl.kernel(
        _sc_body,
        compiler_params=pltpu.CompilerParams(use_tc_tiling_on_sc=False),
        out_type=(
            jax.ShapeDtypeStruct((BATCH * FEW * 2, 128), jnp.float32),
            jax.ShapeDtypeStruct((BATCH, 128), jnp.float32),
        ),
        mesh=mesh,
        scratch_types=[
            pltpu.VMEM((BPW, BLK), jnp.int32),
            pltpu.VMEM((G * BLK, 128), jnp.float32),
            pltpu.VMEM((G * BLK, 128), jnp.float32),
            pltpu.VMEM((REL_PER_W,), jnp.int32),
            pltpu.VMEM((REL_PER_W, EMBED_DIM), jnp.float32),
            pltpu.SemaphoreType.DMA,
            pltpu.SemaphoreType.DMA,
            pltpu.SemaphoreType.DMA,
        ],
    )
    return kfn(idx2d, idxR1d, t128, rel_table)


def kernel(idx, idxR, ent_table, rel_table):
    idx2d = idx.reshape(BATCH, FEW * 2).astype(jnp.int32)
    idxR1d = idxR.reshape(BATCH).astype(jnp.int32)
    ent128, rel128 = _run(idx2d, idxR1d, ent_table, rel_table)
    ent_emb = ent128[:, :EMBED_DIM].reshape(BATCH, FEW, 2, EMBED_DIM)
    rel_emb = rel128[:, :EMBED_DIM].reshape(BATCH, 1, 1, EMBED_DIM)
    return (ent_emb, rel_emb)


# speedup vs baseline: 2.0606x; 1.0376x over previous
import jax
import jax.numpy as jnp
from jax import lax
from jax.experimental import pallas as pl
from jax.experimental.pallas import tpu as pltpu
from jax.experimental.pallas import tpu_sc as plsc

NUM_ENT = 1000000
NUM_REL = 1000
EMBED_DIM = 64
BATCH = 4096
FEW = 50

NC = 2
NS = 16
NW = NC * NS

BLK = FEW * 2
BPW = BATCH // NW
G = 4
NSB = BPW // G
REL_PER_W = BATCH // NW

TBLK = 16384
TGRID = (NUM_ENT + TBLK - 1) // TBLK


def _tc_transpose_body(tab_t_ref, out_ref):
    out_ref[:, :EMBED_DIM] = tab_t_ref[...].T


def _tc_transpose(tab_t):
    return pl.pallas_call(
        _tc_transpose_body,
        grid=(TGRID,),
        in_specs=[pl.BlockSpec((EMBED_DIM, TBLK), lambda j: (0, j))],
        out_specs=pl.BlockSpec((TBLK, 128), lambda j: (j, 0)),
        out_shape=jax.ShapeDtypeStruct((NUM_ENT, 128), jnp.float32),
    )(tab_t)


def _sc_body(idx_hbm, idxR_hbm, ent_hbm, rel_hbm, ent_out, rel_out,
             idx_v, rows_a, rows_b, ridx_v, rrows_v, sem_a, sem_b, sem_r):
    wid = lax.axis_index("s") * NC + lax.axis_index("c")
    b0w = wid * BPW

    pltpu.sync_copy(idxR_hbm.at[pl.ds(b0w, REL_PER_W)], ridx_v)
    rel_dma = pltpu.async_copy(rel_hbm.at[ridx_v], rrows_v, sem_r)

    pltpu.sync_copy(idx_hbm.at[pl.ds(b0w, BPW)], idx_v)

    rel_dma.wait()
    pltpu.sync_copy(rrows_v, rel_out.at[pl.ds(b0w, REL_PER_W), pl.ds(0, EMBED_DIM)])

    def fire(buf, sem, sb):
        for k in range(G):
            pltpu.async_copy(ent_hbm.at[idx_v.at[sb * G + k]],
                             buf.at[pl.ds(k * BLK, BLK)], sem)

    def drain(buf, sem, sb):
        for k in range(G):
            pltpu.make_async_copy(ent_hbm.at[idx_v.at[sb * G + k]],
                                  buf.at[pl.ds(k * BLK, BLK)], sem).wait()
        pltpu.sync_copy(
            buf.at[:, pl.ds(0, EMBED_DIM)],
            ent_out.at[pl.ds((b0w + sb * G) * BLK, G * BLK), pl.ds(0, EMBED_DIM)])

    fire(rows_a, sem_a, 0)

    def pair(j2, carry):
        s0 = 2 * j2
        fire(rows_b, sem_b, s0 + 1)
        drain(rows_a, sem_a, s0)

        @pl.when(s0 + 2 < NSB)
        def _():
            fire(rows_a, sem_a, s0 + 2)

        drain(rows_b, sem_b, s0 + 1)
        return carry

    lax.fori_loop(0, NSB // 2, pair, None)


@jax.jit
def _run(idx2d, idxR1d, ent_table, rel_table):
    t128 = _tc_transpose(ent_table.T)
    mesh = plsc.VectorSubcoreMesh(core_axis_name="c", subcore_axis_name="s",
                                  num_cores=NC, num_subcores=NS)
    kfn = pl.kernel(
        _sc_body,
        compiler_params=pltpu.CompilerParams(use_tc_tiling_on_sc=False),
        out_type=(
            jax.ShapeDtypeStruct((BATCH * FEW * 2, 128), jnp.float32),
            jax.ShapeDtypeStruct((BATCH, 128), jnp.float32),
        ),
        mesh=mesh,
        scratch_types=[
            pltpu.VMEM((BPW, BLK), jnp.int32),
            pltpu.VMEM((G * BLK, 128), jnp.float32),
            pltpu.VMEM((G * BLK, 128), jnp.float32),
            pltpu.VMEM((REL_PER_W,), jnp.int32),
            pltpu.VMEM((REL_PER_W, EMBED_DIM), jnp.float32),
            pltpu.SemaphoreType.DMA,
            pltpu.SemaphoreType.DMA,
            pltpu.SemaphoreType.DMA,
        ],
    )
    return kfn(idx2d, idxR1d, t128, rel_table)


def kernel(idx, idxR, ent_table, rel_table):
    idx2d = idx.reshape(BATCH, FEW * 2).astype(jnp.int32)
    idxR1d = idxR.reshape(BATCH).astype(jnp.int32)
    ent128, rel128 = _run(idx2d, idxR1d, ent_table, rel_table)
    ent_emb = ent128[:, :EMBED_DIM].reshape(BATCH, FEW, 2, EMBED_DIM)
    rel_emb = rel128[:, :EMBED_DIM].reshape(BATCH, 1, 1, EMBED_DIM)
    return (ent_emb, rel_emb)


# speedup vs baseline: 2.0893x; 1.0140x over previous
import jax
import jax.numpy as jnp
from jax import lax
from jax.experimental import pallas as pl
from jax.experimental.pallas import tpu as pltpu
from jax.experimental.pallas import tpu_sc as plsc

NUM_ENT = 1000000
NUM_REL = 1000
EMBED_DIM = 64
BATCH = 4096
FEW = 50

NC = 2
NS = 16
NW = NC * NS

BLK = FEW * 2
BPW = BATCH // NW
G = 4
NSB = BPW // G
REL_PER_W = BATCH // NW

TBLK = 32768
TGRID = (NUM_ENT + TBLK - 1) // TBLK


def _tc_transpose_body(tab_t_ref, out_ref):
    out_ref[:, :EMBED_DIM] = tab_t_ref[...].T


def _tc_transpose(tab_t):
    return pl.pallas_call(
        _tc_transpose_body,
        grid=(TGRID,),
        in_specs=[pl.BlockSpec((EMBED_DIM, TBLK), lambda j: (0, j))],
        out_specs=pl.BlockSpec((TBLK, 128), lambda j: (j, 0)),
        out_shape=jax.ShapeDtypeStruct((NUM_ENT, 128), jnp.float32),
    )(tab_t)


def _sc_body(idx_hbm, idxR_hbm, ent_hbm, rel_hbm, ent_out, rel_out,
             idx_v, rows_a, rows_b, ridx_v, rrows_v, sem_a, sem_b, sem_r):
    wid = lax.axis_index("s") * NC + lax.axis_index("c")
    b0w = wid * BPW

    pltpu.sync_copy(idxR_hbm.at[pl.ds(b0w, REL_PER_W)], ridx_v)
    rel_dma = pltpu.async_copy(rel_hbm.at[ridx_v], rrows_v, sem_r)

    pltpu.sync_copy(idx_hbm.at[pl.ds(b0w, BPW)], idx_v)

    rel_dma.wait()
    pltpu.sync_copy(rrows_v, rel_out.at[pl.ds(b0w, REL_PER_W), pl.ds(0, EMBED_DIM)])

    def fire(buf, sem, sb):
        for k in range(G):
            pltpu.async_copy(ent_hbm.at[idx_v.at[sb * G + k]],
                             buf.at[pl.ds(k * BLK, BLK)], sem)

    def drain(buf, sem, sb):
        for k in range(G):
            pltpu.make_async_copy(ent_hbm.at[idx_v.at[sb * G + k]],
                                  buf.at[pl.ds(k * BLK, BLK)], sem).wait()
        pltpu.sync_copy(
            buf.at[:, pl.ds(0, EMBED_DIM)],
            ent_out.at[pl.ds((b0w + sb * G) * BLK, G * BLK), pl.ds(0, EMBED_DIM)])

    fire(rows_a, sem_a, 0)

    def pair(j2, carry):
        s0 = 2 * j2
        fire(rows_b, sem_b, s0 + 1)
        drain(rows_a, sem_a, s0)

        @pl.when(s0 + 2 < NSB)
        def _():
            fire(rows_a, sem_a, s0 + 2)

        drain(rows_b, sem_b, s0 + 1)
        return carry

    lax.fori_loop(0, NSB // 2, pair, None)


@jax.jit
def _run(idx2d, idxR1d, ent_table, rel_table):
    t128 = _tc_transpose(ent_table.T)
    mesh = plsc.VectorSubcoreMesh(core_axis_name="c", subcore_axis_name="s",
                                  num_cores=NC, num_subcores=NS)
    kfn = pl.kernel(
        _sc_body,
        compiler_params=pltpu.CompilerParams(use_tc_tiling_on_sc=False),
        out_type=(
            jax.ShapeDtypeStruct((BATCH * FEW * 2, 128), jnp.float32),
            jax.ShapeDtypeStruct((BATCH, 128), jnp.float32),
        ),
        mesh=mesh,
        scratch_types=[
            pltpu.VMEM((BPW, BLK), jnp.int32),
            pltpu.VMEM((G * BLK, 128), jnp.float32),
            pltpu.VMEM((G * BLK, 128), jnp.float32),
            pltpu.VMEM((REL_PER_W,), jnp.int32),
            pltpu.VMEM((REL_PER_W, EMBED_DIM), jnp.float32),
            pltpu.SemaphoreType.DMA,
            pltpu.SemaphoreType.DMA,
            pltpu.SemaphoreType.DMA,
        ],
    )
    return kfn(idx2d, idxR1d, t128, rel_table)


def kernel(idx, idxR, ent_table, rel_table):
    idx2d = idx.reshape(BATCH, FEW * 2).astype(jnp.int32)
    idxR1d = idxR.reshape(BATCH).astype(jnp.int32)
    ent128, rel128 = _run(idx2d, idxR1d, ent_table, rel_table)
    ent_emb = ent128[:, :EMBED_DIM].reshape(BATCH, FEW, 2, EMBED_DIM)
    rel_emb = rel128[:, :EMBED_DIM].reshape(BATCH, 1, 1, EMBED_DIM)
    return (ent_emb, rel_emb)
